# baseline (speedup 1.0000x reference)
import functools

import numpy as np
import jax
import jax.numpy as jnp
from jax import lax
from jax.experimental import pallas as pl
from jax.experimental.pallas import tpu as pltpu
from jax.experimental.pallas import tpu_sc as plsc

N = 16384
SQ = 128
_DISTINCT = 64
_MAX_GROUPS = int(1.5 * _DISTINCT)
NGROUPS = min(int(np.random.default_rng(0).integers(1, _MAX_GROUPS + 1)), N)
_BASE, _REM = divmod(N, NGROUPS)

_M32 = 0xFFFFFFFF
_ROTS = ((13, 15, 26, 6), (17, 29, 16, 24))


def _tf2x32_host(k0, k1, x0, x1):
    ks = (k0, k1, (k0 ^ k1 ^ 0x1BD11BDA) & _M32)
    x0 = (x0 + ks[0]) & _M32
    x1 = (x1 + ks[1]) & _M32
    for i in range(5):
        for r in _ROTS[i % 2]:
            x0 = (x0 + x1) & _M32
            x1 = ((x1 << r) | (x1 >> (32 - r))) & _M32
            x1 ^= x0
        x0 = (x0 + ks[(i + 1) % 3]) & _M32
        x1 = (x1 + ks[(i + 2) % 3] + i + 1) & _M32
    return x0, x1


def _derive_round_keys():
    key = _tf2x32_host(0, 0, 0, 1)
    out = []
    for _ in range(2):
        new_key = _tf2x32_host(key[0], key[1], 0, 0)
        subkey = _tf2x32_host(key[0], key[1], 0, 1)
        key = new_key
        out.append(subkey)
    return out


_SK1, _SK2 = _derive_round_keys()


def _tf2x32_vec(k0, k1, cnt):
    ks = (jnp.uint32(k0), jnp.uint32(k1),
          jnp.uint32((k0 ^ k1 ^ 0x1BD11BDA) & _M32))
    x0 = jnp.full_like(cnt, ks[0])
    x1 = cnt + ks[1]
    for i in range(5):
        for r in _ROTS[i % 2]:
            x0 = x0 + x1
            x1 = (x1 << jnp.uint32(r)) | (x1 >> jnp.uint32(32 - r))
            x1 = x1 ^ x0
        x0 = x0 + ks[(i + 1) % 3]
        x1 = x1 + ks[(i + 2) % 3] + jnp.uint32(i + 1)
    return x0 ^ x1


def _keygen_body(o1_ref, o2_ref):
    row = lax.broadcasted_iota(jnp.uint32, (SQ, SQ), 0)
    col = lax.broadcasted_iota(jnp.uint32, (SQ, SQ), 1)
    cnt = row * jnp.uint32(SQ) + col
    for (k0, k1), oref in ((_SK1, o1_ref), (_SK2, o2_ref)):
        bits = _tf2x32_vec(k0, k1, cnt)
        oref[...] = lax.bitcast_convert_type(
            bits ^ jnp.uint32(0x80000000), jnp.int32)


def _rank_body(k1_ref, k2_ref, k1c_ref, k2c_ref, r1_ref, r2_ref):
    ki1 = k1c_ref[0]
    ki2 = k2c_ref[0]

    def body(b, accs):
        a1, a2 = accs
        row1 = k1_ref[pl.ds(b, 1), :]
        row2 = k2_ref[pl.ds(b, 1), :]
        a1 = a1 + (row1 < ki1).astype(jnp.int32)
        a2 = a2 + (row2 < ki2).astype(jnp.int32)
        return a1, a2

    z = jnp.zeros((SQ, SQ), jnp.int32)
    a1, a2 = lax.fori_loop(0, SQ, body, (z, z))
    r1_ref[0] = jnp.sum(a1, axis=1, keepdims=True)
    r2_ref[0] = jnp.sum(a2, axis=1, keepdims=True)


_OH_BLK = 2048
_OH_GRID = N // _OH_BLK


def _onehot_body(j_ref, s_ref, e_ref, o_ref):
    jrow = j_ref[0]
    s = s_ref[...]
    e = e_ref[...]
    o_ref[...] = ((jrow >= s) & (jrow < e)).astype(jnp.float32)


def _sc_compose():
    info = plsc.get_sparse_core_info()
    nc, ns, nl = info.num_cores, info.num_subcores, info.num_lanes
    nw = nc * ns
    chunk = N // nw
    mesh = plsc.VectorSubcoreMesh(core_axis_name="c", subcore_axis_name="s")

    @functools.partial(
        pl.kernel,
        out_type=jax.ShapeDtypeStruct((N,), jnp.int32),
        mesh=mesh,
        compiler_params=pltpu.CompilerParams(needs_layout_passes=False),
        scratch_types=[
            pltpu.VMEM((N,), jnp.int32),
            pltpu.VMEM((chunk,), jnp.int32),
            pltpu.VMEM((chunk,), jnp.int32),
        ],
    )
    def compose(r1_hbm, r2_hbm, out_hbm, r2_v, r1_v, out_v):
        wid = lax.axis_index("s") * nc + lax.axis_index("c")
        base = wid * chunk
        pltpu.sync_copy(r2_hbm, r2_v)
        pltpu.sync_copy(r1_hbm.at[pl.ds(base, chunk)], r1_v)

        def body(i, carry):
            idx = r1_v[pl.ds(i * nl, nl)]
            out_v[pl.ds(i * nl, nl)] = plsc.load_gather(r2_v, [idx])
            return carry

        lax.fori_loop(0, chunk // nl, body, 0)
        pltpu.sync_copy(out_v, out_hbm.at[pl.ds(base, chunk)])

    return compose


def _group_bounds():
    sizes = np.array([_BASE + 1] * _REM + [_BASE] * (NGROUPS - _REM), np.int64)
    starts = np.zeros(NGROUPS, np.int64)
    starts[1:] = np.cumsum(sizes)[:-1]
    ends = starts + sizes
    return (jnp.asarray(starts.reshape(NGROUPS, 1), jnp.int32),
            jnp.asarray(ends.reshape(NGROUPS, 1), jnp.int32))


def kernel(x):
    del x
    k1, k2 = pl.pallas_call(
        _keygen_body,
        out_shape=[jax.ShapeDtypeStruct((SQ, SQ), jnp.int32)] * 2,
    )()

    r1, r2 = pl.pallas_call(
        _rank_body,
        grid=(SQ,),
        in_specs=[
            pl.BlockSpec((SQ, SQ), lambda a: (0, 0)),
            pl.BlockSpec((SQ, SQ), lambda a: (0, 0)),
            pl.BlockSpec((1, SQ, 1), lambda a: (a, 0, 0)),
            pl.BlockSpec((1, SQ, 1), lambda a: (a, 0, 0)),
        ],
        out_specs=[pl.BlockSpec((1, SQ, 1), lambda a: (a, 0, 0))] * 2,
        out_shape=[jax.ShapeDtypeStruct((SQ, SQ, 1), jnp.int32)] * 2,
    )(k1, k2, k1.reshape(SQ, SQ, 1), k2.reshape(SQ, SQ, 1))

    j = _sc_compose()(r1.reshape(N), r2.reshape(N))

    starts, ends = _group_bounds()
    masks = pl.pallas_call(
        _onehot_body,
        grid=(_OH_GRID,),
        in_specs=[
            pl.BlockSpec((1, 1, _OH_BLK), lambda g: (g, 0, 0)),
            pl.BlockSpec((NGROUPS, 1), lambda g: (0, 0)),
            pl.BlockSpec((NGROUPS, 1), lambda g: (0, 0)),
        ],
        out_specs=pl.BlockSpec((NGROUPS, _OH_BLK), lambda g: (0, g)),
        out_shape=jax.ShapeDtypeStruct((NGROUPS, N), jnp.float32),
    )(j.reshape(_OH_GRID, 1, _OH_BLK), starts, ends)
    return masks


# speedup vs baseline: 21.7788x; 21.7788x over previous
import functools

import numpy as np
import jax
import jax.numpy as jnp
from jax import lax
from jax.experimental import pallas as pl
from jax.experimental.pallas import tpu as pltpu
from jax.experimental.pallas import tpu_sc as plsc

N = 16384
SQ = 128
_DISTINCT = 64
_MAX_GROUPS = int(1.5 * _DISTINCT)
NGROUPS = min(int(np.random.default_rng(0).integers(1, _MAX_GROUPS + 1)), N)
_BASE, _REM = divmod(N, NGROUPS)

_M32 = 0xFFFFFFFF
_ROTS = ((13, 15, 26, 6), (17, 29, 16, 24))


def _tf2x32_host(k0, k1, x0, x1):
    ks = (k0, k1, (k0 ^ k1 ^ 0x1BD11BDA) & _M32)
    x0 = (x0 + ks[0]) & _M32
    x1 = (x1 + ks[1]) & _M32
    for i in range(5):
        for r in _ROTS[i % 2]:
            x0 = (x0 + x1) & _M32
            x1 = ((x1 << r) | (x1 >> (32 - r))) & _M32
            x1 ^= x0
        x0 = (x0 + ks[(i + 1) % 3]) & _M32
        x1 = (x1 + ks[(i + 2) % 3] + i + 1) & _M32
    return x0, x1


def _derive_round_keys():
    key = _tf2x32_host(0, 0, 0, 1)
    out = []
    for _ in range(2):
        new_key = _tf2x32_host(key[0], key[1], 0, 0)
        subkey = _tf2x32_host(key[0], key[1], 0, 1)
        key = new_key
        out.append(subkey)
    return out


_SK1, _SK2 = _derive_round_keys()


def _group_ids():
    sizes = [_BASE + 1] * _REM + [_BASE] * (NGROUPS - _REM)
    return np.repeat(np.arange(NGROUPS), sizes).astype(np.int32)


def _tf2x32_vec(k0, k1, cnt):
    ks = (jnp.uint32(k0), jnp.uint32(k1),
          jnp.uint32((k0 ^ k1 ^ 0x1BD11BDA) & _M32))
    x0 = jnp.full_like(cnt, ks[0])
    x1 = cnt + ks[1]
    for i in range(5):
        for r in _ROTS[i % 2]:
            x0 = x0 + x1
            x1 = (x1 << jnp.uint32(r)) | (x1 >> jnp.uint32(32 - r))
            x1 = x1 ^ x0
        x0 = x0 + ks[(i + 1) % 3]
        x1 = x1 + ks[(i + 2) % 3] + jnp.uint32(i + 1)
    return x0 ^ x1


def _signed_keys(k0, k1, cnt):
    bits = _tf2x32_vec(k0, k1, cnt)
    return lax.bitcast_convert_type(bits ^ jnp.uint32(0x80000000), jnp.int32)


def _bitonic_pass(key, val, row_io, col_io):
    k = 2
    while k <= N:
        if k < SQ:
            up = (col_io & k) == 0
        elif k < N:
            up = (row_io & (k // SQ)) == 0
        else:
            up = None
        d = k // 2
        while d >= 1:
            if d < SQ:
                side = (col_io & d) == 0
                fwd_k = pltpu.roll(key, SQ - d, 1)
                bwd_k = pltpu.roll(key, d, 1)
                fwd_v = pltpu.roll(val, SQ - d, 1)
                bwd_v = pltpu.roll(val, d, 1)
            else:
                e = d // SQ
                side = (row_io & e) == 0
                fwd_k = pltpu.roll(key, SQ - e, 0)
                bwd_k = pltpu.roll(key, e, 0)
                fwd_v = pltpu.roll(val, SQ - e, 0)
                bwd_v = pltpu.roll(val, e, 0)
            pk = jnp.where(side, fwd_k, bwd_k)
            pv = jnp.where(side, fwd_v, bwd_v)
            take_small = side if up is None else (up == side)
            take_partner = (pk < key) == take_small
            key = jnp.where(take_partner, pk, key)
            val = jnp.where(take_partner, pv, val)
            d //= 2
        k *= 2
    return key, val


def _sort_body(r_ref):
    row_io = lax.broadcasted_iota(jnp.int32, (SQ, SQ), 0)
    col_io = lax.broadcasted_iota(jnp.int32, (SQ, SQ), 1)
    cnt = (lax.broadcasted_iota(jnp.uint32, (SQ, SQ), 0) * jnp.uint32(SQ)
           + lax.broadcasted_iota(jnp.uint32, (SQ, SQ), 1))
    val = row_io * SQ + col_io
    key1 = _signed_keys(_SK1[0], _SK1[1], cnt)
    _, val = _bitonic_pass(key1, val, row_io, col_io)
    key2 = _signed_keys(_SK2[0], _SK2[1], cnt)
    _, val = _bitonic_pass(key2, val, row_io, col_io)
    r_ref[...] = val


def _sc_scatter():
    info = plsc.get_sparse_core_info()
    nc, ns = info.num_cores, info.num_subcores
    nw = nc * ns
    rows_per = SQ // nw
    mesh = plsc.VectorSubcoreMesh(core_axis_name="c", subcore_axis_name="s")

    @functools.partial(
        pl.kernel,
        out_type=jax.ShapeDtypeStruct((N,), jnp.int32),
        mesh=mesh,
        compiler_params=pltpu.CompilerParams(needs_layout_passes=False),
        scratch_types=[
            pltpu.VMEM((rows_per, SQ), jnp.int32),
            pltpu.VMEM((rows_per, SQ), jnp.int32),
            pltpu.SemaphoreType.DMA,
        ],
    )
    def scat(r_hbm, g_hbm, out_hbm, r_v, g_v, sem):
        wid = lax.axis_index("s") * nc + lax.axis_index("c")
        base = wid * rows_per
        pltpu.sync_copy(r_hbm.at[pl.ds(base, rows_per), :], r_v)
        pltpu.sync_copy(g_hbm.at[pl.ds(base, rows_per), :], g_v)
        copies = [
            pltpu.async_copy(g_v.at[q], out_hbm.at[r_v.at[q]], sem)
            for q in range(rows_per)
        ]
        for c in copies:
            c.wait()

    return scat


_OH_BLK = 2048
_OH_GRID = N // _OH_BLK


def _onehot_body(j_ref, o_ref):
    jrow = j_ref[0]
    gio = lax.broadcasted_iota(jnp.int32, (NGROUPS, _OH_BLK), 0)
    o_ref[...] = (jrow == gio).astype(jnp.float32)


def kernel(x):
    del x
    r = pl.pallas_call(
        _sort_body,
        out_shape=jax.ShapeDtypeStruct((SQ, SQ), jnp.int32),
    )()

    g_const = jnp.asarray(_group_ids().reshape(SQ, SQ))
    gid = _sc_scatter()(r, g_const)

    masks = pl.pallas_call(
        _onehot_body,
        grid=(_OH_GRID,),
        in_specs=[pl.BlockSpec((1, 1, _OH_BLK), lambda g: (g, 0, 0))],
        out_specs=pl.BlockSpec((NGROUPS, _OH_BLK), lambda g: (0, g)),
        out_shape=jax.ShapeDtypeStruct((NGROUPS, N), jnp.float32),
    )(gid.reshape(_OH_GRID, 1, _OH_BLK))
    return masks


# speedup vs baseline: 54.5744x; 2.5059x over previous
import functools

import numpy as np
import jax
import jax.numpy as jnp
from jax import lax
from jax.experimental import pallas as pl
from jax.experimental.pallas import tpu as pltpu
from jax.experimental.pallas import tpu_sc as plsc

N = 16384
SQ = 128
_DISTINCT = 64
_MAX_GROUPS = int(1.5 * _DISTINCT)
NGROUPS = min(int(np.random.default_rng(0).integers(1, _MAX_GROUPS + 1)), N)
_BASE, _REM = divmod(N, NGROUPS)

_M32 = 0xFFFFFFFF
_ROTS = ((13, 15, 26, 6), (17, 29, 16, 24))


def _tf2x32_host(k0, k1, x0, x1):
    ks = (k0, k1, (k0 ^ k1 ^ 0x1BD11BDA) & _M32)
    x0 = (x0 + ks[0]) & _M32
    x1 = (x1 + ks[1]) & _M32
    for i in range(5):
        for r in _ROTS[i % 2]:
            x0 = (x0 + x1) & _M32
            x1 = ((x1 << r) | (x1 >> (32 - r))) & _M32
            x1 ^= x0
        x0 = (x0 + ks[(i + 1) % 3]) & _M32
        x1 = (x1 + ks[(i + 2) % 3] + i + 1) & _M32
    return x0, x1


def _derive_round_keys():
    key = _tf2x32_host(0, 0, 0, 1)
    out = []
    for _ in range(2):
        new_key = _tf2x32_host(key[0], key[1], 0, 0)
        subkey = _tf2x32_host(key[0], key[1], 0, 1)
        key = new_key
        out.append(subkey)
    return out


_SK1, _SK2 = _derive_round_keys()


def _group_ids():
    sizes = [_BASE + 1] * _REM + [_BASE] * (NGROUPS - _REM)
    return np.repeat(np.arange(NGROUPS), sizes).astype(np.int32)


def _tf2x32_vec(k0, k1, cnt):
    ks = (jnp.uint32(k0), jnp.uint32(k1),
          jnp.uint32((k0 ^ k1 ^ 0x1BD11BDA) & _M32))
    x0 = jnp.full_like(cnt, ks[0])
    x1 = cnt + ks[1]
    for i in range(5):
        for r in _ROTS[i % 2]:
            x0 = x0 + x1
            x1 = (x1 << jnp.uint32(r)) | (x1 >> jnp.uint32(32 - r))
            x1 = x1 ^ x0
        x0 = x0 + ks[(i + 1) % 3]
        x1 = x1 + ks[(i + 2) % 3] + jnp.uint32(i + 1)
    return x0 ^ x1


def _signed_keys(k0, k1, cnt):
    bits = _tf2x32_vec(k0, k1, cnt)
    return lax.bitcast_convert_type(bits ^ jnp.uint32(0x80000000), jnp.int32)


def _bitonic_pass(key, val, row_io, col_io):
    k = 2
    while k <= N:
        if k < SQ:
            up = (col_io & k) == 0
        elif k < N:
            up = (row_io & (k // SQ)) == 0
        else:
            up = None
        d = k // 2
        while d >= 1:
            if d < SQ:
                side = (col_io & d) == 0
                fwd_k = pltpu.roll(key, SQ - d, 1)
                bwd_k = pltpu.roll(key, d, 1)
                fwd_v = pltpu.roll(val, SQ - d, 1)
                bwd_v = pltpu.roll(val, d, 1)
            else:
                e = d // SQ
                side = (row_io & e) == 0
                fwd_k = pltpu.roll(key, SQ - e, 0)
                bwd_k = pltpu.roll(key, e, 0)
                fwd_v = pltpu.roll(val, SQ - e, 0)
                bwd_v = pltpu.roll(val, e, 0)
            pk = jnp.where(side, fwd_k, bwd_k)
            pv = jnp.where(side, fwd_v, bwd_v)
            take_small = side if up is None else (up == side)
            take_partner = (pk < key) == take_small
            key = jnp.where(take_partner, pk, key)
            val = jnp.where(take_partner, pv, val)
            d //= 2
        k *= 2
    return key, val


def _sort_body(g_ref, gid_ref):
    row_io = lax.broadcasted_iota(jnp.int32, (SQ, SQ), 0)
    col_io = lax.broadcasted_iota(jnp.int32, (SQ, SQ), 1)
    cnt = (lax.broadcasted_iota(jnp.uint32, (SQ, SQ), 0) * jnp.uint32(SQ)
           + lax.broadcasted_iota(jnp.uint32, (SQ, SQ), 1))
    val = row_io * SQ + col_io
    key1 = _signed_keys(_SK1[0], _SK1[1], cnt)
    _, val = _bitonic_pass(key1, val, row_io, col_io)
    key2 = _signed_keys(_SK2[0], _SK2[1], cnt)
    _, r = _bitonic_pass(key2, val, row_io, col_io)
    _, gid = _bitonic_pass(r, g_ref[...], row_io, col_io)
    gid_ref[...] = gid


def _sc_scatter():
    info = plsc.get_sparse_core_info()
    nc, ns = info.num_cores, info.num_subcores
    nw = nc * ns
    rows_per = SQ // nw
    mesh = plsc.VectorSubcoreMesh(core_axis_name="c", subcore_axis_name="s")

    @functools.partial(
        pl.kernel,
        out_type=jax.ShapeDtypeStruct((N,), jnp.int32),
        mesh=mesh,
        compiler_params=pltpu.CompilerParams(needs_layout_passes=False),
        scratch_types=[
            pltpu.VMEM((rows_per, SQ), jnp.int32),
            pltpu.VMEM((rows_per, SQ), jnp.int32),
            pltpu.SemaphoreType.DMA,
        ],
    )
    def scat(r_hbm, g_hbm, out_hbm, r_v, g_v, sem):
        wid = lax.axis_index("s") * nc + lax.axis_index("c")
        base = wid * rows_per
        pltpu.sync_copy(r_hbm.at[pl.ds(base, rows_per), :], r_v)
        pltpu.sync_copy(g_hbm.at[pl.ds(base, rows_per), :], g_v)
        copies = [
            pltpu.async_copy(g_v.at[q], out_hbm.at[r_v.at[q]], sem)
            for q in range(rows_per)
        ]
        for c in copies:
            c.wait()

    return scat


_OH_BLK = 2048
_OH_GRID = N // _OH_BLK


def _onehot_body(j_ref, o_ref):
    jrow = j_ref[0]
    gio = lax.broadcasted_iota(jnp.int32, (NGROUPS, _OH_BLK), 0)
    o_ref[...] = (jrow == gio).astype(jnp.float32)


def kernel(x):
    del x
    g_const = jnp.asarray(_group_ids().reshape(SQ, SQ))
    gid = pl.pallas_call(
        _sort_body,
        out_shape=jax.ShapeDtypeStruct((SQ, SQ), jnp.int32),
    )(g_const)

    masks = pl.pallas_call(
        _onehot_body,
        grid=(_OH_GRID,),
        in_specs=[pl.BlockSpec((1, 1, _OH_BLK), lambda g: (g, 0, 0))],
        out_specs=pl.BlockSpec((NGROUPS, _OH_BLK), lambda g: (0, g)),
        out_shape=jax.ShapeDtypeStruct((NGROUPS, N), jnp.float32),
    )(gid.reshape(_OH_GRID, 1, _OH_BLK))
    return masks
